# baseline (speedup 1.0000x reference)
import jax
import jax.numpy as jnp
from jax import lax
from jax.experimental import pallas as pl
from jax.experimental.pallas import tpu as pltpu
from jax.experimental.pallas import tpu_sc as plsc

N = 10000
E = 320000
G = 64
NPAD = 10240
EPA_ROWS = 2560
EPA = EPA_ROWS * 128
BW = 112
EPB_BLKS = 960
NB0 = 37
NB1 = 23
EPB = EPB_BLKS * 3 * BW
BLK = 512
NSTEPS = NPAD // BLK
SRC_PAD = 10000
DST_PAD = 10001

_mesh = plsc.VectorSubcoreMesh(core_axis_name="c", subcore_axis_name="s")
_sc_params = pltpu.CompilerParams(needs_layout_passes=False)


def _sc_a_body(src2d, dst2d, batchp, xp,
               dinv_out, xt_out, u_out, cnt_out,
               deg_s, dinv_s, u_s, cnt_s,
               srcring, dstring, flatring, valring,
               dinvbuf, batchbuf, xtbuf, degbuf, dvbuf, cntbuf):
    cid = lax.axis_index("c")
    sid = lax.axis_index("s")
    wid = cid * 16 + sid
    nb = wid * 320
    ones16 = jnp.ones((16,), jnp.float32)
    zeros16 = jnp.zeros((16,), jnp.float32)
    iota16 = lax.iota(jnp.int32, 16)

    def f0(i, c):
        degbuf[pl.ds(i * 16, 16)] = ones16
        return c
    lax.fori_loop(0, 40, f0, 0)

    def f1(i, c):
        dvbuf[pl.ds(i * 16, 16)] = zeros16
        return c
    lax.fori_loop(0, 40, f1, 0)

    pltpu.sync_copy(degbuf, deg_s.at[pl.ds(sid * 640, 640)])

    def f1b(m, c):
        pltpu.sync_copy(dvbuf, u_s.at[pl.ds(sid * 40960 + m * 640, 640)])
        return c
    lax.fori_loop(0, 64, f1b, 0)

    @pl.when(sid == 0)
    def _():
        def fz(i, c):
            cntbuf[pl.ds(i * 16, 16)] = zeros16
            return c
        lax.fori_loop(0, 4, fz, 0)
        pltpu.sync_copy(cntbuf, cnt_s)

    def f2(k, c):
        valring[0, pl.ds(k * 16, 16)] = ones16
        return c
    lax.fori_loop(0, 8, f2, 0)

    plsc.subcore_barrier()

    def fg(g, c):
        row0 = sid * 160 + g * 16
        pltpu.sync_copy(dst2d.at[pl.ds(row0, 16)], dstring)

        def fj(j, c2):
            pltpu.sync_copy(valring.at[0], deg_s.at[dstring.at[j]], add=True)
            return c2
        lax.fori_loop(0, 16, fj, 0)
        return c
    lax.fori_loop(0, 10, fg, 0)
    plsc.subcore_barrier()

    pltpu.sync_copy(deg_s.at[pl.ds(sid * 640, 640)], degbuf)

    def fr(i, c):
        d = degbuf[pl.ds(i * 16, 16)]
        y = jnp.ones((16,), jnp.float32)
        for _ in range(10):
            y = jnp.where(d * y * y > 1.0, y * 0.5, y)
        for _ in range(5):
            y = y * (1.5 - 0.5 * d * y * y)
        dvbuf[pl.ds(i * 16, 16)] = y
        return c
    lax.fori_loop(0, 40, fr, 0)

    pltpu.sync_copy(dvbuf, dinv_s.at[pl.ds(sid * 640, 640)])

    @pl.when(cid == 0)
    def _():
        pltpu.sync_copy(dvbuf, dinv_out.at[pl.ds(sid * 640, 640)])

    plsc.subcore_barrier()

    pltpu.sync_copy(dinv_s, dinvbuf)
    pltpu.sync_copy(batchp, batchbuf)

    def fm(i, c):
        dinvbuf[pl.ds(N + i * 16, 16)] = zeros16
        return c
    lax.fori_loop(0, (NPAD - N) // 16, fm, 0)

    pltpu.sync_copy(xp.at[pl.ds(nb, 320)], xtbuf)

    def fx(r, c):
        idx = jnp.zeros((16,), jnp.int32) + (nb + r)
        dv = plsc.load_gather(dinvbuf, [idx])

        def fk(k, c2):
            xtbuf[r, pl.ds(k * 16, 16)] = xtbuf[r, pl.ds(k * 16, 16)] * dv
            return c2
        lax.fori_loop(0, 8, fk, 0)
        return c
    lax.fori_loop(0, 320, fx, 0)

    pltpu.sync_copy(xtbuf, xt_out.at[pl.ds(nb, 320)])

    def ug(g, c):
        row0 = wid * 80 + g * 16
        pltpu.sync_copy(src2d.at[pl.ds(row0, 16)], srcring)
        pltpu.sync_copy(dst2d.at[pl.ds(row0, 16)], dstring)

        def uj(j, c2):
            def uk(k, c3):
                sv = srcring[j, pl.ds(k * 16, 16)]
                dvec = dstring[j, pl.ds(k * 16, 16)]
                dd = plsc.load_gather(dinvbuf, [dvec])
                bd = plsc.load_gather(batchbuf, [dvec])
                flatring[j, pl.ds(k * 16, 16)] = sv * G + bd
                valring[j, pl.ds(k * 16, 16)] = dd
                return c3
            lax.fori_loop(0, 8, uk, 0)
            pltpu.sync_copy(valring.at[j], u_s.at[flatring.at[j]], add=True)
            return c2
        lax.fori_loop(0, 16, uj, 0)
        return c
    lax.fori_loop(0, 5, ug, 0)

    def sg(m, c):
        off = m * 16 + iota16
        n16 = jnp.minimum(nb + off, NPAD - 1)
        inb = off < 320
        val = plsc.load_gather(dinvbuf, [n16])
        b16 = plsc.load_gather(batchbuf, [n16])
        row = m // 8
        col = (m % 8) * 16
        flatring[row, pl.ds(col, 16)] = n16 * G + b16
        valring[row, pl.ds(col, 16)] = jnp.where(inb, val, 0.0)
        return c
    lax.fori_loop(0, 24, sg, 0)

    def sd(row, c):
        pltpu.sync_copy(valring.at[row], u_s.at[flatring.at[row]], add=True)
        return c
    lax.fori_loop(0, 3, sd, 0)

    def cg(m, c):
        off = m * 16 + iota16
        raw = nb + off
        n16 = jnp.minimum(raw, NPAD - 1)
        b16 = plsc.load_gather(batchbuf, [n16])
        ok = jnp.logical_and(off < 320, raw < N)
        valring[m // 8, pl.ds((m % 8) * 16, 16)] = jnp.where(ok, 1.0, 0.0)
        flatring[m // 8, pl.ds((m % 8) * 16, 16)] = b16
        return c
    lax.fori_loop(0, 24, cg, 0)

    def cd(row, c):
        pltpu.sync_copy(valring.at[row], cnt_s.at[flatring.at[row]], add=True)
        return c
    lax.fori_loop(0, 3, cd, 0)

    plsc.subcore_barrier()

    pltpu.sync_copy(u_s.at[pl.ds(sid * 40960, 40960)],
                    u_out.at[cid, pl.ds(sid * 40960, 40960)])

    @pl.when(sid == 0)
    def _():
        pltpu.sync_copy(cnt_s, cnt_out.at[cid])


_sc_a = pl.kernel(
    _sc_a_body,
    out_type=[
        jax.ShapeDtypeStruct((NPAD,), jnp.float32),
        jax.ShapeDtypeStruct((NPAD, 128), jnp.float32),
        jax.ShapeDtypeStruct((2, NPAD * G), jnp.float32),
        jax.ShapeDtypeStruct((2, G), jnp.float32),
    ],
    mesh=_mesh,
    compiler_params=_sc_params,
    scratch_types=[
        pltpu.VMEM_SHARED((NPAD,), jnp.float32),
        pltpu.VMEM_SHARED((NPAD,), jnp.float32),
        pltpu.VMEM_SHARED((NPAD * G,), jnp.float32),
        pltpu.VMEM_SHARED((G,), jnp.float32),
        pltpu.VMEM((16, 128), jnp.int32),
        pltpu.VMEM((16, 128), jnp.int32),
        pltpu.VMEM((16, 128), jnp.int32),
        pltpu.VMEM((16, 128), jnp.float32),
        pltpu.VMEM((NPAD,), jnp.float32),
        pltpu.VMEM((NPAD,), jnp.int32),
        pltpu.VMEM((320, 128), jnp.float32),
        pltpu.VMEM((640,), jnp.float32),
        pltpu.VMEM((640,), jnp.float32),
        pltpu.VMEM((64,), jnp.float32),
    ],
)


def _sc_b_body(src3d, dst3d, xt, aggp,
               agg_s, ring_s, ring_d, g0, g1, g2,
               sg0, sg1, sg2, ss0, ss1, ss2):
    cid = lax.axis_index("c")
    sid = lax.axis_index("s")
    wid = cid * 16 + sid
    nblk = jnp.where(cid == 0, NB0, NB1)
    base = jnp.where(cid == 0, sid * NB0, 16 * NB0 + sid * NB1)
    zeros16 = jnp.zeros((16,), jnp.float32)
    bufs = (g0, g1, g2)
    gsems = (sg0, sg1, sg2)
    ssems = (ss0, ss1, ss2)

    @pl.when(cid == 0)
    def _():
        pltpu.sync_copy(xt.at[pl.ds(sid * 640, 640)],
                        agg_s.at[pl.ds(sid * 640, 640)])

    @pl.when(cid == 1)
    def _():
        def fz(r, c):
            def fk(k, c2):
                g0[r, pl.ds(k * 16, 16)] = zeros16
                return c2
            lax.fori_loop(0, 8, fk, 0)
            return c
        lax.fori_loop(0, BW, fz, 0)

        def fcp(m, c):
            pltpu.sync_copy(g0, agg_s.at[pl.ds(sid * 640 + m * BW, BW)])
            return c
        lax.fori_loop(0, 5, fcp, 0)
        pltpu.sync_copy(g0.at[pl.ds(0, 80)],
                        agg_s.at[pl.ds(sid * 640 + 560, 80)])

    plsc.subcore_barrier()

    def grp(g, c):
        blk = base + g
        h = g % 2
        pltpu.sync_copy(src3d.at[blk], ring_s.at[h])
        pltpu.sync_copy(dst3d.at[blk], ring_d.at[h])
        for b in range(3):
            @pl.when(g > 0)
            def _():
                pltpu.make_async_copy(bufs[b], agg_s.at[ring_d.at[h, b]],
                                      ssems[b]).wait()
            pltpu.async_copy(xt.at[ring_s.at[h, b]], bufs[b], gsems[b])
        for b in range(3):
            pltpu.make_async_copy(xt.at[ring_s.at[h, b]], bufs[b],
                                  gsems[b]).wait()
            pltpu.async_copy(bufs[b], agg_s.at[ring_d.at[h, b]], ssems[b],
                             add=True)
        return c
    lax.fori_loop(0, nblk, grp, 0)

    for b in range(3):
        pltpu.make_async_copy(bufs[b], agg_s.at[ring_d.at[1, b]],
                              ssems[b]).wait()

    plsc.subcore_barrier()
    pltpu.sync_copy(agg_s.at[pl.ds(sid * 640, 640)],
                    aggp.at[cid, pl.ds(sid * 640, 640)])


_sc_b = pl.kernel(
    _sc_b_body,
    out_type=[jax.ShapeDtypeStruct((2, NPAD, 128), jnp.float32)],
    mesh=_mesh,
    compiler_params=_sc_params,
    scratch_types=[
        pltpu.VMEM_SHARED((NPAD, 128), jnp.float32),
        pltpu.VMEM((2, 3, BW), jnp.int32),
        pltpu.VMEM((2, 3, BW), jnp.int32),
        pltpu.VMEM((BW, 128), jnp.float32),
        pltpu.VMEM((BW, 128), jnp.float32),
        pltpu.VMEM((BW, 128), jnp.float32),
        pltpu.SemaphoreType.DMA,
        pltpu.SemaphoreType.DMA,
        pltpu.SemaphoreType.DMA,
        pltpu.SemaphoreType.DMA,
        pltpu.SemaphoreType.DMA,
        pltpu.SemaphoreType.DMA,
    ],
)


def _tc_body(agg_ref, dinv_ref, u_ref, cnt_ref, w1_ref, b1_ref, w2_ref,
             b2_ref, wc_ref, bc_ref, out_ref, s_acc):
    k = pl.program_id(0)

    @pl.when(k == 0)
    def _():
        s_acc[...] = jnp.zeros_like(s_acc)

    dv = dinv_ref[...]
    agg = dv * (agg_ref[0] + agg_ref[1])
    h1 = jnp.maximum(jnp.dot(agg, w1_ref[...],
                             preferred_element_type=jnp.float32,
                             precision=lax.Precision.HIGHEST)
                     + b1_ref[...], 0.0)
    uw = dv * (u_ref[0] + u_ref[1])
    s_acc[...] += jax.lax.dot_general(
        uw, h1, (((0,), (0,)), ((), ())),
        preferred_element_type=jnp.float32,
        precision=lax.Precision.HIGHEST)

    @pl.when(k == NSTEPS - 1)
    def _():
        cnt = cnt_ref[0] + cnt_ref[1]
        pooled = jnp.dot(s_acc[...] / cnt, w2_ref[...],
                         preferred_element_type=jnp.float32,
                         precision=lax.Precision.HIGHEST) + b2_ref[...]
        out_ref[...] = jnp.dot(pooled, wc_ref[...],
                               preferred_element_type=jnp.float32,
                               precision=lax.Precision.HIGHEST) + bc_ref[...]


def _tc_finish(aggp, dinv_p, u_p, counts, W1, b1, W2, b2, Wc, bc):
    hid = W1.shape[1]
    c = Wc.shape[1]
    return pl.pallas_call(
        _tc_body,
        grid=(NSTEPS,),
        in_specs=[
            pl.BlockSpec((2, BLK, 128), lambda k: (0, k, 0)),
            pl.BlockSpec((BLK, 1), lambda k: (k, 0)),
            pl.BlockSpec((2, BLK, G), lambda k: (0, k, 0)),
            pl.BlockSpec((2, G, 1), lambda k: (0, 0, 0)),
            pl.BlockSpec((128, hid), lambda k: (0, 0)),
            pl.BlockSpec((1, hid), lambda k: (0, 0)),
            pl.BlockSpec((hid, hid), lambda k: (0, 0)),
            pl.BlockSpec((1, hid), lambda k: (0, 0)),
            pl.BlockSpec((hid, c), lambda k: (0, 0)),
            pl.BlockSpec((1, c), lambda k: (0, 0)),
        ],
        out_specs=pl.BlockSpec((G, c), lambda k: (0, 0)),
        out_shape=jax.ShapeDtypeStruct((G, c), jnp.float32),
        scratch_shapes=[pltpu.VMEM((G, hid), jnp.float32)],
    )(aggp, dinv_p, u_p, counts, W1, b1, W2, b2, Wc, bc)


@jax.jit
def kernel(x, edge_index, batch, W1, b1, W2, b2, Wc, bc):
    src = edge_index[0]
    dst = edge_index[1]
    srcpa = jnp.concatenate(
        [src, jnp.full((EPA - E,), SRC_PAD, jnp.int32)]).reshape(EPA_ROWS, 128)
    dstpa = jnp.concatenate(
        [dst, jnp.full((EPA - E,), DST_PAD, jnp.int32)]).reshape(EPA_ROWS, 128)
    srcpb = jnp.concatenate(
        [src, jnp.full((EPB - E,), SRC_PAD, jnp.int32)]).reshape(EPB_BLKS, 3,
                                                                 BW)
    dstpb = jnp.concatenate(
        [dst, jnp.full((EPB - E,), DST_PAD, jnp.int32)]).reshape(EPB_BLKS, 3,
                                                                 BW)
    batchp = jnp.concatenate(
        [batch, jnp.full((NPAD - N,), G - 1, jnp.int32)])
    xp = jnp.zeros((NPAD, 128), jnp.float32).at[:N].set(x)

    dinv, xt, u2, cnt2 = _sc_a(srcpa, dstpa, batchp, xp)
    (aggp,) = _sc_b(srcpb, dstpb, xt)

    return _tc_finish(aggp, dinv.reshape(NPAD, 1), u2.reshape(2, NPAD, G),
                      cnt2.reshape(2, G, 1),
                      W1, b1[None, :], W2, b2[None, :], Wc, bc[None, :])


# speedup vs baseline: 1.0191x; 1.0191x over previous
import jax
import jax.numpy as jnp
from jax import lax
from jax.experimental import pallas as pl
from jax.experimental.pallas import tpu as pltpu
from jax.experimental.pallas import tpu_sc as plsc

N = 10000
E = 320000
G = 64
NPAD = 10240
EPA_ROWS = 2560
EPA = EPA_ROWS * 128
BW = 112
EPB_BLKS = 960
NB0 = 37
NB1 = 23
EPB = EPB_BLKS * 3 * BW
BLK = 512
NSTEPS = NPAD // BLK
SRC_PAD = 10000
DST_PAD = 10001

_mesh = plsc.VectorSubcoreMesh(core_axis_name="c", subcore_axis_name="s")
_sc_params = pltpu.CompilerParams(needs_layout_passes=False)


def _sc_a_body(src2d, dst2d, batchp, xp,
               dinv_out, xt_out, u_out, cnt_out,
               deg_s, dinv_s, u_s, cnt_s,
               srcring, dstring, flatring, valring,
               dinvbuf, batchbuf, xtbuf, degbuf, dvbuf, cntbuf):
    cid = lax.axis_index("c")
    sid = lax.axis_index("s")
    wid = cid * 16 + sid
    nb = wid * 320
    ones16 = jnp.ones((16,), jnp.float32)
    zeros16 = jnp.zeros((16,), jnp.float32)
    iota16 = lax.iota(jnp.int32, 16)

    def f0(i, c):
        degbuf[pl.ds(i * 16, 16)] = ones16
        return c
    lax.fori_loop(0, 40, f0, 0)

    def f1(i, c):
        dvbuf[pl.ds(i * 16, 16)] = zeros16
        return c
    lax.fori_loop(0, 40, f1, 0)

    pltpu.sync_copy(degbuf, deg_s.at[pl.ds(sid * 640, 640)])

    def f1b(m, c):
        pltpu.sync_copy(dvbuf, u_s.at[pl.ds(sid * 40960 + m * 640, 640)])
        return c
    lax.fori_loop(0, 64, f1b, 0)

    @pl.when(sid == 0)
    def _():
        def fz(i, c):
            cntbuf[pl.ds(i * 16, 16)] = zeros16
            return c
        lax.fori_loop(0, 4, fz, 0)
        pltpu.sync_copy(cntbuf, cnt_s)

    def f2(k, c):
        valring[0, pl.ds(k * 16, 16)] = ones16
        return c
    lax.fori_loop(0, 8, f2, 0)

    plsc.subcore_barrier()

    def fg(g, c):
        row0 = sid * 160 + g * 16
        pltpu.sync_copy(dst2d.at[pl.ds(row0, 16)], dstring)

        def fj(j, c2):
            pltpu.sync_copy(valring.at[0], deg_s.at[dstring.at[j]], add=True)
            return c2
        lax.fori_loop(0, 16, fj, 0)
        return c
    lax.fori_loop(0, 10, fg, 0)
    plsc.subcore_barrier()

    pltpu.sync_copy(deg_s.at[pl.ds(sid * 640, 640)], degbuf)

    def fr(i, c):
        d = degbuf[pl.ds(i * 16, 16)]
        y = jnp.ones((16,), jnp.float32)
        for _ in range(10):
            y = jnp.where(d * y * y > 1.0, y * 0.5, y)
        for _ in range(5):
            y = y * (1.5 - 0.5 * d * y * y)
        dvbuf[pl.ds(i * 16, 16)] = y
        return c
    lax.fori_loop(0, 40, fr, 0)

    pltpu.sync_copy(dvbuf, dinv_s.at[pl.ds(sid * 640, 640)])

    @pl.when(cid == 0)
    def _():
        pltpu.sync_copy(dvbuf, dinv_out.at[pl.ds(sid * 640, 640)])

    plsc.subcore_barrier()

    pltpu.sync_copy(dinv_s, dinvbuf)
    pltpu.sync_copy(batchp, batchbuf)

    def fm(i, c):
        dinvbuf[pl.ds(N + i * 16, 16)] = zeros16
        return c
    lax.fori_loop(0, (NPAD - N) // 16, fm, 0)

    pltpu.sync_copy(xp.at[pl.ds(nb, 320)], xtbuf)

    def fx(r, c):
        idx = jnp.zeros((16,), jnp.int32) + (nb + r)
        dv = plsc.load_gather(dinvbuf, [idx])

        def fk(k, c2):
            xtbuf[r, pl.ds(k * 16, 16)] = xtbuf[r, pl.ds(k * 16, 16)] * dv
            return c2
        lax.fori_loop(0, 8, fk, 0)
        return c
    lax.fori_loop(0, 320, fx, 0)

    pltpu.sync_copy(xtbuf, xt_out.at[pl.ds(nb, 320)])

    def ug(g, c):
        row0 = wid * 80 + g * 16
        pltpu.sync_copy(src2d.at[pl.ds(row0, 16)], srcring)
        pltpu.sync_copy(dst2d.at[pl.ds(row0, 16)], dstring)

        def uj(j, c2):
            def uk(k, c3):
                sv = srcring[j, pl.ds(k * 16, 16)]
                dvec = dstring[j, pl.ds(k * 16, 16)]
                dd = plsc.load_gather(dinvbuf, [dvec])
                bd = plsc.load_gather(batchbuf, [dvec])
                flatring[j, pl.ds(k * 16, 16)] = sv * G + bd
                valring[j, pl.ds(k * 16, 16)] = dd
                return c3
            lax.fori_loop(0, 8, uk, 0)
            pltpu.sync_copy(valring.at[j], u_s.at[flatring.at[j]], add=True)
            return c2
        lax.fori_loop(0, 16, uj, 0)
        return c
    lax.fori_loop(0, 5, ug, 0)

    def sg(m, c):
        off = m * 16 + iota16
        n16 = jnp.minimum(nb + off, NPAD - 1)
        inb = off < 320
        val = plsc.load_gather(dinvbuf, [n16])
        b16 = plsc.load_gather(batchbuf, [n16])
        row = m // 8
        col = (m % 8) * 16
        flatring[row, pl.ds(col, 16)] = n16 * G + b16
        valring[row, pl.ds(col, 16)] = jnp.where(inb, val, 0.0)
        return c
    lax.fori_loop(0, 24, sg, 0)

    def sd(row, c):
        pltpu.sync_copy(valring.at[row], u_s.at[flatring.at[row]], add=True)
        return c
    lax.fori_loop(0, 3, sd, 0)

    def cg(m, c):
        off = m * 16 + iota16
        raw = nb + off
        n16 = jnp.minimum(raw, NPAD - 1)
        b16 = plsc.load_gather(batchbuf, [n16])
        ok = jnp.logical_and(off < 320, raw < N)
        valring[m // 8, pl.ds((m % 8) * 16, 16)] = jnp.where(ok, 1.0, 0.0)
        flatring[m // 8, pl.ds((m % 8) * 16, 16)] = b16
        return c
    lax.fori_loop(0, 24, cg, 0)

    def cd(row, c):
        pltpu.sync_copy(valring.at[row], cnt_s.at[flatring.at[row]], add=True)
        return c
    lax.fori_loop(0, 3, cd, 0)

    plsc.subcore_barrier()

    pltpu.sync_copy(u_s.at[pl.ds(sid * 40960, 40960)],
                    u_out.at[cid, pl.ds(sid * 40960, 40960)])

    @pl.when(sid == 0)
    def _():
        pltpu.sync_copy(cnt_s, cnt_out.at[cid])


_sc_a = pl.kernel(
    _sc_a_body,
    out_type=[
        jax.ShapeDtypeStruct((NPAD,), jnp.float32),
        jax.ShapeDtypeStruct((NPAD, 128), jnp.float32),
        jax.ShapeDtypeStruct((2, NPAD * G), jnp.float32),
        jax.ShapeDtypeStruct((2, G), jnp.float32),
    ],
    mesh=_mesh,
    compiler_params=_sc_params,
    scratch_types=[
        pltpu.VMEM_SHARED((NPAD,), jnp.float32),
        pltpu.VMEM_SHARED((NPAD,), jnp.float32),
        pltpu.VMEM_SHARED((NPAD * G,), jnp.float32),
        pltpu.VMEM_SHARED((G,), jnp.float32),
        pltpu.VMEM((16, 128), jnp.int32),
        pltpu.VMEM((16, 128), jnp.int32),
        pltpu.VMEM((16, 128), jnp.int32),
        pltpu.VMEM((16, 128), jnp.float32),
        pltpu.VMEM((NPAD,), jnp.float32),
        pltpu.VMEM((NPAD,), jnp.int32),
        pltpu.VMEM((320, 128), jnp.float32),
        pltpu.VMEM((640,), jnp.float32),
        pltpu.VMEM((640,), jnp.float32),
        pltpu.VMEM((64,), jnp.float32),
    ],
)


def _sc_b_body(src3d, dst3d, xt, aggp,
               agg_s, ring_s, ring_d, g0, g1, g2,
               sg0, sg1, sg2, ss0, ss1, ss2):
    cid = lax.axis_index("c")
    sid = lax.axis_index("s")
    wid = cid * 16 + sid
    nblk = jnp.where(cid == 0, NB0, NB1)
    base = jnp.where(cid == 0, sid * NB0, 16 * NB0 + sid * NB1)
    zeros16 = jnp.zeros((16,), jnp.float32)
    bufs = (g0, g1, g2)
    gsems = (sg0, sg1, sg2)
    ssems = (ss0, ss1, ss2)

    @pl.when(cid == 0)
    def _():
        pltpu.sync_copy(xt.at[pl.ds(sid * 640, 640)],
                        agg_s.at[pl.ds(sid * 640, 640)])

    @pl.when(cid == 1)
    def _():
        def fz(r, c):
            def fk(k, c2):
                g0[r, pl.ds(k * 16, 16)] = zeros16
                return c2
            lax.fori_loop(0, 8, fk, 0)
            return c
        lax.fori_loop(0, BW, fz, 0)

        def fcp(m, c):
            pltpu.sync_copy(g0, agg_s.at[pl.ds(sid * 640 + m * BW, BW)])
            return c
        lax.fori_loop(0, 5, fcp, 0)
        pltpu.sync_copy(g0.at[pl.ds(0, 80)],
                        agg_s.at[pl.ds(sid * 640 + 560, 80)])

    plsc.subcore_barrier()

    def grp(g, c):
        blk = base + g
        h = g % 2
        pltpu.sync_copy(src3d.at[blk], ring_s.at[h])
        pltpu.sync_copy(dst3d.at[blk], ring_d.at[h])
        for b in range(3):
            @pl.when(g > 0)
            def _():
                pltpu.make_async_copy(bufs[b], agg_s.at[ring_d.at[h, b]],
                                      ssems[b]).wait()
            pltpu.async_copy(xt.at[ring_s.at[h, b]], bufs[b], gsems[b])
        for b in range(3):
            pltpu.make_async_copy(xt.at[ring_s.at[h, b]], bufs[b],
                                  gsems[b]).wait()
            pltpu.async_copy(bufs[b], agg_s.at[ring_d.at[h, b]], ssems[b],
                             add=True)
        return c
    lax.fori_loop(0, nblk, grp, 0)

    for b in range(3):
        pltpu.make_async_copy(bufs[b], agg_s.at[ring_d.at[1, b]],
                              ssems[b]).wait()

    plsc.subcore_barrier()
    pltpu.sync_copy(agg_s.at[pl.ds(sid * 640, 640)],
                    aggp.at[cid, pl.ds(sid * 640, 640)])


_sc_b = pl.kernel(
    _sc_b_body,
    out_type=[jax.ShapeDtypeStruct((2, NPAD, 128), jnp.float32)],
    mesh=_mesh,
    compiler_params=_sc_params,
    scratch_types=[
        pltpu.VMEM_SHARED((NPAD, 128), jnp.float32),
        pltpu.VMEM((2, 3, BW), jnp.int32),
        pltpu.VMEM((2, 3, BW), jnp.int32),
        pltpu.VMEM((BW, 128), jnp.float32),
        pltpu.VMEM((BW, 128), jnp.float32),
        pltpu.VMEM((BW, 128), jnp.float32),
        pltpu.SemaphoreType.DMA,
        pltpu.SemaphoreType.DMA,
        pltpu.SemaphoreType.DMA,
        pltpu.SemaphoreType.DMA,
        pltpu.SemaphoreType.DMA,
        pltpu.SemaphoreType.DMA,
    ],
)


def _split3(a):
    hi = a.astype(jnp.bfloat16).astype(jnp.float32)
    return hi, a - hi


def _mm3(a, b):
    a_hi, a_lo = _split3(a)
    b_hi, b_lo = _split3(b)
    f = jnp.float32
    return (jnp.dot(a_hi, b_hi, preferred_element_type=f)
            + jnp.dot(a_hi, b_lo, preferred_element_type=f)
            + jnp.dot(a_lo, b_hi, preferred_element_type=f))


def _mm3_t(a, b):
    dims = (((0,), (0,)), ((), ()))
    a_hi, a_lo = _split3(a)
    b_hi, b_lo = _split3(b)
    f = jnp.float32
    return (lax.dot_general(a_hi, b_hi, dims, preferred_element_type=f)
            + lax.dot_general(a_hi, b_lo, dims, preferred_element_type=f)
            + lax.dot_general(a_lo, b_hi, dims, preferred_element_type=f))


def _tc_body(agg_ref, dinv_ref, u_ref, cnt_ref, w1_ref, b1_ref, w2_ref,
             b2_ref, wc_ref, bc_ref, out_ref, s_acc):
    k = pl.program_id(0)

    @pl.when(k == 0)
    def _():
        s_acc[...] = jnp.zeros_like(s_acc)

    dv = dinv_ref[...]
    agg = dv * (agg_ref[0] + agg_ref[1])
    h1 = jnp.maximum(_mm3(agg, w1_ref[...])
                     + b1_ref[...], 0.0)
    uw = dv * (u_ref[0] + u_ref[1])
    s_acc[...] += _mm3_t(uw, h1)

    @pl.when(k == NSTEPS - 1)
    def _():
        cnt = cnt_ref[0] + cnt_ref[1]
        pooled = (s_acc[...] / cnt) @ w2_ref[...] + b2_ref[...]
        out_ref[...] = pooled @ wc_ref[...] + bc_ref[...]


def _tc_finish(aggp, dinv_p, u_p, counts, W1, b1, W2, b2, Wc, bc):
    hid = W1.shape[1]
    c = Wc.shape[1]
    return pl.pallas_call(
        _tc_body,
        grid=(NSTEPS,),
        in_specs=[
            pl.BlockSpec((2, BLK, 128), lambda k: (0, k, 0)),
            pl.BlockSpec((BLK, 1), lambda k: (k, 0)),
            pl.BlockSpec((2, BLK, G), lambda k: (0, k, 0)),
            pl.BlockSpec((2, G, 1), lambda k: (0, 0, 0)),
            pl.BlockSpec((128, hid), lambda k: (0, 0)),
            pl.BlockSpec((1, hid), lambda k: (0, 0)),
            pl.BlockSpec((hid, hid), lambda k: (0, 0)),
            pl.BlockSpec((1, hid), lambda k: (0, 0)),
            pl.BlockSpec((hid, c), lambda k: (0, 0)),
            pl.BlockSpec((1, c), lambda k: (0, 0)),
        ],
        out_specs=pl.BlockSpec((G, c), lambda k: (0, 0)),
        out_shape=jax.ShapeDtypeStruct((G, c), jnp.float32),
        scratch_shapes=[pltpu.VMEM((G, hid), jnp.float32)],
    )(aggp, dinv_p, u_p, counts, W1, b1, W2, b2, Wc, bc)


@jax.jit
def kernel(x, edge_index, batch, W1, b1, W2, b2, Wc, bc):
    src = edge_index[0]
    dst = edge_index[1]
    srcpa = jnp.concatenate(
        [src, jnp.full((EPA - E,), SRC_PAD, jnp.int32)]).reshape(EPA_ROWS, 128)
    dstpa = jnp.concatenate(
        [dst, jnp.full((EPA - E,), DST_PAD, jnp.int32)]).reshape(EPA_ROWS, 128)
    srcpb = jnp.concatenate(
        [src, jnp.full((EPB - E,), SRC_PAD, jnp.int32)]).reshape(EPB_BLKS, 3,
                                                                 BW)
    dstpb = jnp.concatenate(
        [dst, jnp.full((EPB - E,), DST_PAD, jnp.int32)]).reshape(EPB_BLKS, 3,
                                                                 BW)
    batchp = jnp.concatenate(
        [batch, jnp.full((NPAD - N,), G - 1, jnp.int32)])
    xp = jnp.zeros((NPAD, 128), jnp.float32).at[:N].set(x)

    dinv, xt, u2, cnt2 = _sc_a(srcpa, dstpa, batchp, xp)
    (aggp,) = _sc_b(srcpb, dstpb, xt)

    return _tc_finish(aggp, dinv.reshape(NPAD, 1), u2.reshape(2, NPAD, G),
                      cnt2.reshape(2, G, 1),
                      W1, b1[None, :], W2, b2[None, :], Wc, bc[None, :])


# speedup vs baseline: 1.0527x; 1.0330x over previous
import jax
import jax.numpy as jnp
from jax import lax
from jax.experimental import pallas as pl
from jax.experimental.pallas import tpu as pltpu
from jax.experimental.pallas import tpu_sc as plsc

N = 10000
E = 320000
G = 64
NPAD = 10240
EPA_ROWS = 2560
EPA = EPA_ROWS * 128
BW = 112
EPB_BLKS = 960
NB0 = 37
NB1 = 23
EPB = EPB_BLKS * 3 * BW
BLK = 512
NSTEPS = NPAD // BLK
SRC_PAD = 10000
DST_PAD = 10001

_mesh = plsc.VectorSubcoreMesh(core_axis_name="c", subcore_axis_name="s")
_sc_params = pltpu.CompilerParams(needs_layout_passes=False)


def _sc_a_body(src2d, dst2d, batchp, xp,
               dinv_out, xt_out, u_out, cnt_out,
               deg_s, dinv_s, u_s, cnt_s,
               srcring, dstring, flatring, valring,
               dinvbuf, batchbuf, xtbuf, degbuf, dvbuf, cntbuf,
               sa0, sa1):
    cid = lax.axis_index("c")
    sid = lax.axis_index("s")
    wid = cid * 16 + sid
    nb = wid * 320
    ones16 = jnp.ones((16,), jnp.float32)
    zeros16 = jnp.zeros((16,), jnp.float32)
    iota16 = lax.iota(jnp.int32, 16)

    def f0(i, c):
        degbuf[pl.ds(i * 16, 16)] = ones16
        return c
    lax.fori_loop(0, 40, f0, 0)

    def f1(i, c):
        dinvbuf[pl.ds(i * 16, 16)] = zeros16
        return c
    lax.fori_loop(0, 320, f1, 0)

    pltpu.sync_copy(degbuf, deg_s.at[pl.ds(sid * 640, 640)])

    zsrc = dinvbuf.at[pl.ds(0, 5120)]

    def f1b(m, c):
        pltpu.sync_copy(zsrc, u_s.at[pl.ds(sid * 40960 + m * 5120, 5120)])
        return c
    lax.fori_loop(0, 8, f1b, 0)

    @pl.when(sid == 0)
    def _():
        def fz(i, c):
            cntbuf[pl.ds(i * 16, 16)] = zeros16
            return c
        lax.fori_loop(0, 4, fz, 0)
        pltpu.sync_copy(cntbuf, cnt_s)

    def f2(k, c):
        valring[0, pl.ds(k * 16, 16)] = ones16
        return c
    lax.fori_loop(0, 8, f2, 0)

    plsc.subcore_barrier()

    asems = (sa0, sa1)

    def fg(g, c):
        row0 = sid * 160 + g * 8
        h8 = 8 * (g % 2)
        pltpu.sync_copy(dst2d.at[pl.ds(row0, 8)],
                        dstring.at[pl.ds(h8, 8)])
        for b in range(8):
            p = b % 2

            def _wait():
                pltpu.make_async_copy(
                    valring.at[0], deg_s.at[dstring.at[h8 + b]],
                    asems[p]).wait()
            if b < 2:
                pl.when(g > 0)(_wait)
            else:
                _wait()
            pltpu.async_copy(valring.at[0], deg_s.at[dstring.at[h8 + b]],
                             asems[p], add=True)
        return c
    lax.fori_loop(0, 20, fg, 0)
    for p in range(2):
        pltpu.make_async_copy(valring.at[0], deg_s.at[dstring.at[p]],
                              asems[p]).wait()
    plsc.subcore_barrier()

    pltpu.sync_copy(deg_s.at[pl.ds(sid * 640, 640)], degbuf)

    def fr(i, c):
        d = degbuf[pl.ds(i * 16, 16)]
        y = jnp.ones((16,), jnp.float32)
        for _ in range(10):
            y = jnp.where(d * y * y > 1.0, y * 0.5, y)
        for _ in range(5):
            y = y * (1.5 - 0.5 * d * y * y)
        dvbuf[pl.ds(i * 16, 16)] = y
        return c
    lax.fori_loop(0, 40, fr, 0)

    pltpu.sync_copy(dvbuf, dinv_s.at[pl.ds(sid * 640, 640)])

    @pl.when(cid == 0)
    def _():
        pltpu.sync_copy(dvbuf, dinv_out.at[pl.ds(sid * 640, 640)])

    plsc.subcore_barrier()

    pltpu.sync_copy(dinv_s, dinvbuf)
    pltpu.sync_copy(batchp, batchbuf)

    def fm(i, c):
        dinvbuf[pl.ds(N + i * 16, 16)] = zeros16
        return c
    lax.fori_loop(0, (NPAD - N) // 16, fm, 0)

    pltpu.sync_copy(xp.at[pl.ds(nb, 320)], xtbuf)

    def fx(r, c):
        idx = jnp.zeros((16,), jnp.int32) + (nb + r)
        dv = plsc.load_gather(dinvbuf, [idx])

        def fk(k, c2):
            xtbuf[r, pl.ds(k * 16, 16)] = xtbuf[r, pl.ds(k * 16, 16)] * dv
            return c2
        lax.fori_loop(0, 8, fk, 0)
        return c
    lax.fori_loop(0, 320, fx, 0)

    pltpu.sync_copy(xtbuf, xt_out.at[pl.ds(nb, 320)])

    def ug(g, c):
        row0 = wid * 80 + g * 16
        pltpu.sync_copy(src2d.at[pl.ds(row0, 16)], srcring)
        pltpu.sync_copy(dst2d.at[pl.ds(row0, 16)], dstring)
        for j in range(16):
            def uk(k, c3):
                sv = srcring[j, pl.ds(k * 16, 16)]
                dvec = dstring[j, pl.ds(k * 16, 16)]
                dd = plsc.load_gather(dinvbuf, [dvec])
                bd = plsc.load_gather(batchbuf, [dvec])
                flatring[j, pl.ds(k * 16, 16)] = sv * G + bd
                valring[j, pl.ds(k * 16, 16)] = dd
                return c3
            lax.fori_loop(0, 8, uk, 0)
            p = j % 2

            def _wait():
                pltpu.make_async_copy(valring.at[j],
                                      u_s.at[flatring.at[j]],
                                      asems[p]).wait()
            if j < 2:
                pl.when(g > 0)(_wait)
            else:
                _wait()
            pltpu.async_copy(valring.at[j], u_s.at[flatring.at[j]],
                             asems[p], add=True)
        return c
    lax.fori_loop(0, 5, ug, 0)
    for p in range(2):
        pltpu.make_async_copy(valring.at[p], u_s.at[flatring.at[p]],
                              asems[p]).wait()

    def sg(m, c):
        off = m * 16 + iota16
        n16 = jnp.minimum(nb + off, NPAD - 1)
        inb = off < 320
        val = plsc.load_gather(dinvbuf, [n16])
        b16 = plsc.load_gather(batchbuf, [n16])
        row = m // 8
        col = (m % 8) * 16
        flatring[row, pl.ds(col, 16)] = n16 * G + b16
        valring[row, pl.ds(col, 16)] = jnp.where(inb, val, 0.0)
        return c
    lax.fori_loop(0, 24, sg, 0)

    def sd(row, c):
        pltpu.sync_copy(valring.at[row], u_s.at[flatring.at[row]], add=True)
        return c
    lax.fori_loop(0, 3, sd, 0)

    def cg(m, c):
        off = m * 16 + iota16
        raw = nb + off
        n16 = jnp.minimum(raw, NPAD - 1)
        b16 = plsc.load_gather(batchbuf, [n16])
        ok = jnp.logical_and(off < 320, raw < N)
        valring[m // 8, pl.ds((m % 8) * 16, 16)] = jnp.where(ok, 1.0, 0.0)
        flatring[m // 8, pl.ds((m % 8) * 16, 16)] = b16
        return c
    lax.fori_loop(0, 24, cg, 0)

    def cd(row, c):
        pltpu.sync_copy(valring.at[row], cnt_s.at[flatring.at[row]], add=True)
        return c
    lax.fori_loop(0, 3, cd, 0)

    plsc.subcore_barrier()

    pltpu.sync_copy(u_s.at[pl.ds(sid * 40960, 40960)],
                    u_out.at[cid, pl.ds(sid * 40960, 40960)])

    @pl.when(sid == 0)
    def _():
        pltpu.sync_copy(cnt_s, cnt_out.at[cid])


_sc_a = pl.kernel(
    _sc_a_body,
    out_type=[
        jax.ShapeDtypeStruct((NPAD,), jnp.float32),
        jax.ShapeDtypeStruct((NPAD, 128), jnp.float32),
        jax.ShapeDtypeStruct((2, NPAD * G), jnp.float32),
        jax.ShapeDtypeStruct((2, G), jnp.float32),
    ],
    mesh=_mesh,
    compiler_params=_sc_params,
    scratch_types=[
        pltpu.VMEM_SHARED((NPAD,), jnp.float32),
        pltpu.VMEM_SHARED((NPAD,), jnp.float32),
        pltpu.VMEM_SHARED((NPAD * G,), jnp.float32),
        pltpu.VMEM_SHARED((G,), jnp.float32),
        pltpu.VMEM((16, 128), jnp.int32),
        pltpu.VMEM((16, 128), jnp.int32),
        pltpu.VMEM((16, 128), jnp.int32),
        pltpu.VMEM((16, 128), jnp.float32),
        pltpu.VMEM((NPAD,), jnp.float32),
        pltpu.VMEM((NPAD,), jnp.int32),
        pltpu.VMEM((320, 128), jnp.float32),
        pltpu.VMEM((640,), jnp.float32),
        pltpu.VMEM((640,), jnp.float32),
        pltpu.VMEM((64,), jnp.float32),
        pltpu.SemaphoreType.DMA,
        pltpu.SemaphoreType.DMA,
    ],
)


def _sc_b_body(src3d, dst3d, xt, aggp,
               agg_s, ring_s, ring_d, g0, g1, g2,
               sg0, sg1, sg2, ss0, ss1, ss2):
    cid = lax.axis_index("c")
    sid = lax.axis_index("s")
    wid = cid * 16 + sid
    nblk = jnp.where(cid == 0, NB0, NB1)
    base = jnp.where(cid == 0, sid * NB0, 16 * NB0 + sid * NB1)
    zeros16 = jnp.zeros((16,), jnp.float32)
    bufs = (g0, g1, g2)
    gsems = (sg0, sg1, sg2)
    ssems = (ss0, ss1, ss2)

    @pl.when(cid == 0)
    def _():
        pltpu.sync_copy(xt.at[pl.ds(sid * 640, 640)],
                        agg_s.at[pl.ds(sid * 640, 640)])

    @pl.when(cid == 1)
    def _():
        def fz(r, c):
            def fk(k, c2):
                g0[r, pl.ds(k * 16, 16)] = zeros16
                return c2
            lax.fori_loop(0, 8, fk, 0)
            return c
        lax.fori_loop(0, BW, fz, 0)

        def fcp(m, c):
            pltpu.sync_copy(g0, agg_s.at[pl.ds(sid * 640 + m * BW, BW)])
            return c
        lax.fori_loop(0, 5, fcp, 0)
        pltpu.sync_copy(g0.at[pl.ds(0, 80)],
                        agg_s.at[pl.ds(sid * 640 + 560, 80)])

    plsc.subcore_barrier()

    def grp(g, c):
        blk = base + g
        h = g % 2
        pltpu.sync_copy(src3d.at[blk], ring_s.at[h])
        pltpu.sync_copy(dst3d.at[blk], ring_d.at[h])
        for b in range(3):
            @pl.when(g > 0)
            def _():
                pltpu.make_async_copy(bufs[b], agg_s.at[ring_d.at[h, b]],
                                      ssems[b]).wait()
            pltpu.async_copy(xt.at[ring_s.at[h, b]], bufs[b], gsems[b])
        for b in range(3):
            pltpu.make_async_copy(xt.at[ring_s.at[h, b]], bufs[b],
                                  gsems[b]).wait()
            pltpu.async_copy(bufs[b], agg_s.at[ring_d.at[h, b]], ssems[b],
                             add=True)
        return c
    lax.fori_loop(0, nblk, grp, 0)

    for b in range(3):
        pltpu.make_async_copy(bufs[b], agg_s.at[ring_d.at[1, b]],
                              ssems[b]).wait()

    plsc.subcore_barrier()
    pltpu.sync_copy(agg_s.at[pl.ds(sid * 640, 640)],
                    aggp.at[cid, pl.ds(sid * 640, 640)])


_sc_b = pl.kernel(
    _sc_b_body,
    out_type=[jax.ShapeDtypeStruct((2, NPAD, 128), jnp.float32)],
    mesh=_mesh,
    compiler_params=_sc_params,
    scratch_types=[
        pltpu.VMEM_SHARED((NPAD, 128), jnp.float32),
        pltpu.VMEM((2, 3, BW), jnp.int32),
        pltpu.VMEM((2, 3, BW), jnp.int32),
        pltpu.VMEM((BW, 128), jnp.float32),
        pltpu.VMEM((BW, 128), jnp.float32),
        pltpu.VMEM((BW, 128), jnp.float32),
        pltpu.SemaphoreType.DMA,
        pltpu.SemaphoreType.DMA,
        pltpu.SemaphoreType.DMA,
        pltpu.SemaphoreType.DMA,
        pltpu.SemaphoreType.DMA,
        pltpu.SemaphoreType.DMA,
    ],
)


def _tc_body(agg_ref, dinv_ref, u_ref, cnt_ref, w1_ref, b1_ref, w2_ref,
             b2_ref, wc_ref, bc_ref, out_ref, s_acc):
    k = pl.program_id(0)

    @pl.when(k == 0)
    def _():
        s_acc[...] = jnp.zeros_like(s_acc)

    dv = dinv_ref[...]
    agg = dv * (agg_ref[0] + agg_ref[1])
    h1 = jnp.maximum(jnp.dot(agg, w1_ref[...],
                             preferred_element_type=jnp.float32)
                     + b1_ref[...], 0.0)
    uw = dv * (u_ref[0] + u_ref[1])
    s_acc[...] += jax.lax.dot_general(
        uw, h1, (((0,), (0,)), ((), ())),
        preferred_element_type=jnp.float32)

    @pl.when(k == NSTEPS - 1)
    def _():
        cnt = cnt_ref[0] + cnt_ref[1]
        pooled = (s_acc[...] / cnt) @ w2_ref[...] + b2_ref[...]
        out_ref[...] = pooled @ wc_ref[...] + bc_ref[...]


def _tc_finish(aggp, dinv_p, u_p, counts, W1, b1, W2, b2, Wc, bc):
    hid = W1.shape[1]
    c = Wc.shape[1]
    return pl.pallas_call(
        _tc_body,
        grid=(NSTEPS,),
        in_specs=[
            pl.BlockSpec((2, BLK, 128), lambda k: (0, k, 0)),
            pl.BlockSpec((BLK, 1), lambda k: (k, 0)),
            pl.BlockSpec((2, BLK, G), lambda k: (0, k, 0)),
            pl.BlockSpec((2, G, 1), lambda k: (0, 0, 0)),
            pl.BlockSpec((128, hid), lambda k: (0, 0)),
            pl.BlockSpec((1, hid), lambda k: (0, 0)),
            pl.BlockSpec((hid, hid), lambda k: (0, 0)),
            pl.BlockSpec((1, hid), lambda k: (0, 0)),
            pl.BlockSpec((hid, c), lambda k: (0, 0)),
            pl.BlockSpec((1, c), lambda k: (0, 0)),
        ],
        out_specs=pl.BlockSpec((G, c), lambda k: (0, 0)),
        out_shape=jax.ShapeDtypeStruct((G, c), jnp.float32),
        scratch_shapes=[pltpu.VMEM((G, hid), jnp.float32)],
    )(aggp, dinv_p, u_p, counts, W1, b1, W2, b2, Wc, bc)


@jax.jit
def kernel(x, edge_index, batch, W1, b1, W2, b2, Wc, bc):
    src = edge_index[0]
    dst = edge_index[1]
    srcpa = jnp.concatenate(
        [src, jnp.full((EPA - E,), SRC_PAD, jnp.int32)]).reshape(EPA_ROWS, 128)
    dstpa = jnp.concatenate(
        [dst, jnp.full((EPA - E,), DST_PAD, jnp.int32)]).reshape(EPA_ROWS, 128)
    srcpb = jnp.concatenate(
        [src, jnp.full((EPB - E,), SRC_PAD, jnp.int32)]).reshape(EPB_BLKS, 3,
                                                                 BW)
    dstpb = jnp.concatenate(
        [dst, jnp.full((EPB - E,), DST_PAD, jnp.int32)]).reshape(EPB_BLKS, 3,
                                                                 BW)
    batchp = jnp.concatenate(
        [batch, jnp.full((NPAD - N,), G - 1, jnp.int32)])
    xp = jnp.zeros((NPAD, 128), jnp.float32).at[:N].set(x)

    dinv, xt, u2, cnt2 = _sc_a(srcpa, dstpa, batchp, xp)
    (aggp,) = _sc_b(srcpb, dstpb, xt)

    return _tc_finish(aggp, dinv.reshape(NPAD, 1), u2.reshape(2, NPAD, G),
                      cnt2.reshape(2, G, 1),
                      W1, b1[None, :], W2, b2[None, :], Wc, bc[None, :])
